# baseline (speedup 1.0000x reference)
import functools

import jax
import jax.numpy as jnp
from jax import lax
from jax.experimental import pallas as pl
from jax.experimental.pallas import tpu as pltpu
from jax.experimental.pallas import tpu_sc as plsc

D = 16
NC, NS, LANES = 2, 16, 16
NW = NC * NS
CB = 8


def _combo_body(bt, it, ut, wt, bias, c_out):
    code = lax.broadcasted_iota(jnp.int32, (8, 1), 0)
    f1 = (code & 1).astype(jnp.float32)
    f2 = ((code >> 1) & 1).astype(jnp.float32)
    f3 = ((code >> 2) & 1).astype(jnp.float32)
    pb = bt[0:1, :] + f1 * (bt[1:2, :] - bt[0:1, :])
    pi = it[0:1, :] + f2 * (it[1:2, :] - it[0:1, :])
    pu = ut[0:1, :] + f3 * (ut[1:2, :] - ut[0:1, :])
    comb = jnp.concatenate([pb, pi, pu], axis=1)
    c_out[...] = (
        jnp.dot(comb, wt[...], preferred_element_type=jnp.float32) + bias[...]
    )


def _combo_table(bt, it, ut, w_t, bias2d):
    return pl.pallas_call(
        _combo_body,
        out_shape=jax.ShapeDtypeStruct((8, D), jnp.float32),
    )(bt, it, ut, w_t, bias2d)


def _make_sc_lookup(B, T):
    n_tok = B * T
    per_w_b = B // NW
    n_chunk = per_w_b // CB
    chunk_tok = CB * T
    full_g = T // LANES
    tail = T - full_g * LANES
    mesh = plsc.VectorSubcoreMesh(
        core_axis_name="c", subcore_axis_name="s", num_cores=NC, num_subcores=NS
    )

    @functools.partial(
        pl.kernel,
        mesh=mesh,
        compiler_params=pltpu.CompilerParams(use_tc_tiling_on_sc=False),
        out_type=jax.ShapeDtypeStruct((B, T, D), jnp.float32),
        scratch_types=[
            pltpu.VMEM((chunk_tok + LANES,), jnp.int32),
            pltpu.VMEM((chunk_tok + LANES,), jnp.int32),
            pltpu.VMEM((chunk_tok + LANES,), jnp.int32),
            pltpu.VMEM((CB, T, D), jnp.float32),
            pltpu.VMEM((8 * D,), jnp.float32),
            pltpu.SemaphoreType.DMA,
        ],
    )
    def sc_lookup(f1_hbm, f2_hbm, f3_hbm, c_hbm, out_hbm,
                  f1_v, f2_v, f3_v, rows_v, c_v, sem):
        wid = lax.axis_index("s") * NC + lax.axis_index("c")
        b_base = wid * per_w_b
        pltpu.sync_copy(c_hbm, c_v)

        def expand16(off, bi, t0, n):
            a = f1_v[pl.ds(off, LANES)]
            bb = f2_v[pl.ds(off, LANES)]
            cc = f3_v[pl.ds(off, LANES)]
            code16 = a + bb * 2 + cc * 4
            for v in range(n):
                rows_v[bi, t0 + v, :] = c_v[pl.ds(code16[v] * D, D)]

        def chunk_body(ci, carry):
            b0 = b_base + ci * CB
            start = b0 * T
            pltpu.sync_copy(f1_hbm.at[pl.ds(start, chunk_tok)],
                            f1_v.at[pl.ds(0, chunk_tok)])
            pltpu.sync_copy(f2_hbm.at[pl.ds(start, chunk_tok)],
                            f2_v.at[pl.ds(0, chunk_tok)])
            pltpu.sync_copy(f3_hbm.at[pl.ds(start, chunk_tok)],
                            f3_v.at[pl.ds(0, chunk_tok)])

            for bi in range(CB):
                def group_body(g, carry2, bi=bi):
                    expand16(bi * T + g * LANES, bi, g * LANES, LANES)
                    return carry2

                lax.fori_loop(0, full_g, group_body, 0)
                if tail:
                    expand16(bi * T + full_g * LANES, bi, full_g * LANES, tail)

            pltpu.sync_copy(rows_v, out_hbm.at[pl.ds(b0, CB)])
            return carry

        lax.fori_loop(0, n_chunk, chunk_body, 0)

    return sc_lookup


def kernel(bold_flags, italic_flags, underline_flags,
           bold_table, italic_table, underline_table, W, b):
    B, T = bold_flags.shape
    n_tok = B * T
    combo = _combo_table(
        bold_table, italic_table, underline_table,
        W.T, b.reshape(1, D),
    )
    f1 = bold_flags.reshape(n_tok).astype(jnp.int32)
    f2 = italic_flags.reshape(n_tok).astype(jnp.int32)
    f3 = underline_flags.reshape(n_tok).astype(jnp.int32)
    return _make_sc_lookup(B, T)(f1, f2, f3, combo.reshape(8 * D))


# speedup vs baseline: 6.8652x; 6.8652x over previous
import functools

import jax
import jax.numpy as jnp
from jax import lax
from jax.experimental import pallas as pl
from jax.experimental.pallas import tpu as pltpu
from jax.experimental.pallas import tpu_sc as plsc

D = 16
NC, NS, LANES = 2, 16, 16
NW = NC * NS
TB = 25

_GATHER_DNUMS = lax.GatherDimensionNumbers(
    offset_dims=(), collapsed_slice_dims=(0,), start_index_map=(0,)
)


def _combo_body(btT, itT, utT, w, bias, c_out):
    code = lax.broadcasted_iota(jnp.int32, (1, 8), 1)
    f1 = (code & 1).astype(jnp.float32)
    f2 = ((code >> 1) & 1).astype(jnp.float32)
    f3 = ((code >> 2) & 1).astype(jnp.float32)
    pb = btT[:, 0:1] + f1 * (btT[:, 1:2] - btT[:, 0:1])
    pi = itT[:, 0:1] + f2 * (itT[:, 1:2] - itT[:, 0:1])
    pu = utT[:, 0:1] + f3 * (utT[:, 1:2] - utT[:, 0:1])
    combT = jnp.concatenate([pb, pi, pu], axis=0)
    ct = jnp.dot(w[...], combT, preferred_element_type=jnp.float32) + bias[...]
    c_out[...] = jnp.concatenate([ct, jnp.zeros((D, 8), jnp.float32)], axis=1)


def _combo_table_t(btT, itT, utT, w, bias2d):
    return pl.pallas_call(
        _combo_body,
        out_shape=jax.ShapeDtypeStruct((D, D), jnp.float32),
    )(btT, itT, utT, w, bias2d)


def _make_sc_lookup(B, T):
    n_chunk = T // TB
    bt_n = B // 128
    mesh = plsc.VectorSubcoreMesh(
        core_axis_name="c", subcore_axis_name="s", num_cores=NC, num_subcores=NS
    )

    @functools.partial(
        pl.kernel,
        mesh=mesh,
        compiler_params=pltpu.CompilerParams(use_tc_tiling_on_sc=False),
        out_type=jax.ShapeDtypeStruct((T, D // 8, bt_n, 8, 128), jnp.float32),
        scratch_types=[
            pltpu.VMEM((TB, 128), jnp.int32),
            pltpu.VMEM((TB, 128), jnp.int32),
            pltpu.VMEM((TB, 128), jnp.int32),
            pltpu.VMEM((TB, D // 8, 1, 8, 128), jnp.float32),
            pltpu.VMEM((D, D), jnp.float32),
            pltpu.SemaphoreType.DMA,
        ],
    )
    def sc_lookup(f1_hbm, f2_hbm, f3_hbm, ct_hbm, out_hbm,
                  f1_v, f2_v, f3_v, rows_v, ct_v, sem):
        wid = lax.axis_index("s") * NC + lax.axis_index("c")
        b0 = wid * 128
        pltpu.sync_copy(ct_hbm, ct_v)
        cks = [ct_v[k, :] for k in range(D)]

        def chunk_body(ci, carry):
            t0 = ci * TB
            pltpu.sync_copy(f1_hbm.at[pl.ds(t0, TB), pl.ds(b0, 128)], f1_v)
            pltpu.sync_copy(f2_hbm.at[pl.ds(t0, TB), pl.ds(b0, 128)], f2_v)
            pltpu.sync_copy(f3_hbm.at[pl.ds(t0, TB), pl.ds(b0, 128)], f3_v)

            def t_body(lt, carry2):
                for g in range(128 // LANES):
                    s = g * LANES
                    a = f1_v[lt, pl.ds(s, LANES)]
                    bb = f2_v[lt, pl.ds(s, LANES)]
                    cc = f3_v[lt, pl.ds(s, LANES)]
                    code16 = a + bb * 2 + cc * 4
                    for k in range(D):
                        outv = lax.gather(
                            cks[k], code16[:, None], _GATHER_DNUMS, (1,),
                            mode=lax.GatherScatterMode.PROMISE_IN_BOUNDS,
                        )
                        rows_v[lt, k // 8, 0, k % 8, pl.ds(s, LANES)] = outv
                return carry2

            lax.fori_loop(0, TB, t_body, 0)
            pltpu.sync_copy(
                rows_v,
                out_hbm.at[pl.ds(t0, TB), :, pl.ds(wid, 1), :, :],
            )
            return carry

        lax.fori_loop(0, n_chunk, chunk_body, 0)

    return sc_lookup


def kernel(bold_flags, italic_flags, underline_flags,
           bold_table, italic_table, underline_table, W, b):
    B, T = bold_flags.shape
    ct = _combo_table_t(
        bold_table.T, italic_table.T, underline_table.T,
        W, b.reshape(D, 1),
    )
    f1 = bold_flags.T.astype(jnp.int32)
    f2 = italic_flags.T.astype(jnp.int32)
    f3 = underline_flags.T.astype(jnp.int32)
    out5 = _make_sc_lookup(B, T)(f1, f2, f3, ct)
    return out5.transpose((2, 4, 0, 1, 3)).reshape(B, T, D)


# speedup vs baseline: 9.8139x; 1.4295x over previous
import functools

import jax
import jax.numpy as jnp
from jax import lax
from jax.experimental import pallas as pl
from jax.experimental.pallas import tpu as pltpu
from jax.experimental.pallas import tpu_sc as plsc

D = 16
NC, NS, LANES = 2, 16, 16
NW = NC * NS
TB = 25

_GATHER_DNUMS = lax.GatherDimensionNumbers(
    offset_dims=(), collapsed_slice_dims=(0,), start_index_map=(0,)
)


def _combo_body(btT, itT, utT, w, bias, c_out):
    code = lax.broadcasted_iota(jnp.int32, (1, 8), 1)
    f1 = (code & 1).astype(jnp.float32)
    f2 = ((code >> 1) & 1).astype(jnp.float32)
    f3 = ((code >> 2) & 1).astype(jnp.float32)
    pb = btT[:, 0:1] + f1 * (btT[:, 1:2] - btT[:, 0:1])
    pi = itT[:, 0:1] + f2 * (itT[:, 1:2] - itT[:, 0:1])
    pu = utT[:, 0:1] + f3 * (utT[:, 1:2] - utT[:, 0:1])
    combT = jnp.concatenate([pb, pi, pu], axis=0)
    ct = jnp.dot(w[...], combT, preferred_element_type=jnp.float32) + bias[...]
    c_out[...] = jnp.concatenate([ct, jnp.zeros((D, 8), jnp.float32)], axis=1)


def _combo_table_t(btT, itT, utT, w, bias2d):
    return pl.pallas_call(
        _combo_body,
        out_shape=jax.ShapeDtypeStruct((D, D), jnp.float32),
    )(btT, itT, utT, w, bias2d)


def _make_sc_lookup(B, T):
    n_chunk = T // TB
    bt_n = B // 128
    mesh = plsc.VectorSubcoreMesh(
        core_axis_name="c", subcore_axis_name="s", num_cores=NC, num_subcores=NS
    )

    @functools.partial(
        pl.kernel,
        mesh=mesh,
        compiler_params=pltpu.CompilerParams(use_tc_tiling_on_sc=False),
        out_type=jax.ShapeDtypeStruct((T, D // 8, bt_n, 8, 128), jnp.float32),
        scratch_types=[
            pltpu.VMEM((2, TB, 128), jnp.int32),
            pltpu.VMEM((2, TB, 128), jnp.int32),
            pltpu.VMEM((2, TB, 128), jnp.int32),
            pltpu.VMEM((2, TB, D // 8, 1, 8, 128), jnp.float32),
            pltpu.VMEM((D, D), jnp.float32),
            pltpu.SemaphoreType.DMA,
            pltpu.SemaphoreType.DMA,
            pltpu.SemaphoreType.DMA,
            pltpu.SemaphoreType.DMA,
        ],
    )
    def sc_lookup(f1_hbm, f2_hbm, f3_hbm, ct_hbm, out_hbm,
                  f1_v, f2_v, f3_v, rows_v, ct_v,
                  sem_in0, sem_in1, sem_out0, sem_out1):
        wid = lax.axis_index("s") * NC + lax.axis_index("c")
        b0 = wid * 128
        pltpu.sync_copy(ct_hbm, ct_v)
        cks = [ct_v[k, :] for k in range(D)]
        sems_in = (sem_in0, sem_in1)
        sems_out = (sem_out0, sem_out1)

        def start_flags(ci):
            p = ci % 2
            t0 = ci * TB
            return [
                pltpu.async_copy(
                    f_hbm.at[pl.ds(t0, TB), pl.ds(b0, 128)],
                    f_v.at[p], sems_in[p])
                for f_hbm, f_v in ((f1_hbm, f1_v), (f2_hbm, f2_v),
                                   (f3_hbm, f3_v))
            ]

        in_flight = {0: start_flags(0)}
        out_flight = {}
        for ci in range(n_chunk):
            p = ci % 2
            if ci + 1 < n_chunk:
                in_flight[ci + 1] = start_flags(ci + 1)
            for cp in in_flight.pop(ci):
                cp.wait()
            if ci >= 2:
                out_flight.pop(ci - 2).wait()

            def t_body(lt, carry2, p=p):
                for g in range(128 // LANES):
                    s = g * LANES
                    a = f1_v[p, lt, pl.ds(s, LANES)]
                    bb = f2_v[p, lt, pl.ds(s, LANES)]
                    cc = f3_v[p, lt, pl.ds(s, LANES)]
                    code16 = a + bb * 2 + cc * 4
                    for k in range(D):
                        outv = lax.gather(
                            cks[k], code16[:, None], _GATHER_DNUMS, (1,),
                            mode=lax.GatherScatterMode.PROMISE_IN_BOUNDS,
                        )
                        rows_v[p, lt, k // 8, 0, k % 8, pl.ds(s, LANES)] = outv
                return carry2

            lax.fori_loop(0, TB, t_body, 0)
            out_flight[ci] = pltpu.async_copy(
                rows_v.at[p],
                out_hbm.at[pl.ds(ci * TB, TB), :, pl.ds(wid, 1), :, :],
                sems_out[p],
            )
        for cp in out_flight.values():
            cp.wait()

    return sc_lookup


def kernel(bold_flags, italic_flags, underline_flags,
           bold_table, italic_table, underline_table, W, b):
    B, T = bold_flags.shape
    ct = _combo_table_t(
        bold_table.T, italic_table.T, underline_table.T,
        W, b.reshape(D, 1),
    )
    f1 = bold_flags.T.astype(jnp.int32)
    f2 = italic_flags.T.astype(jnp.int32)
    f3 = underline_flags.T.astype(jnp.int32)
    out5 = _make_sc_lookup(B, T)(f1, f2, f3, ct)
    return out5.transpose((2, 4, 0, 1, 3)).reshape(B, T, D)


# speedup vs baseline: 10.6058x; 1.0807x over previous
import functools

import jax
import jax.numpy as jnp
from jax import lax
from jax.experimental import pallas as pl
from jax.experimental.pallas import tpu as pltpu
from jax.experimental.pallas import tpu_sc as plsc

D = 16
NC, NS, LANES = 2, 16, 16
NW = NC * NS
TB = 25

_GATHER_DNUMS = lax.GatherDimensionNumbers(
    offset_dims=(), collapsed_slice_dims=(0,), start_index_map=(0,)
)


def _combo_body(btT, itT, utT, w, bias, c_out):
    code = lax.broadcasted_iota(jnp.int32, (1, 8), 1)
    f1 = (code & 1).astype(jnp.float32)
    f2 = ((code >> 1) & 1).astype(jnp.float32)
    f3 = ((code >> 2) & 1).astype(jnp.float32)
    pb = btT[:, 0:1] + f1 * (btT[:, 1:2] - btT[:, 0:1])
    pi = itT[:, 0:1] + f2 * (itT[:, 1:2] - itT[:, 0:1])
    pu = utT[:, 0:1] + f3 * (utT[:, 1:2] - utT[:, 0:1])
    combT = jnp.concatenate([pb, pi, pu], axis=0)
    ct = jnp.dot(w[...], combT, preferred_element_type=jnp.float32) + bias[...]
    c_out[...] = jnp.concatenate([ct, jnp.zeros((D, 8), jnp.float32)], axis=1)


def _combo_table_t(btT, itT, utT, w, bias2d):
    return pl.pallas_call(
        _combo_body,
        out_shape=jax.ShapeDtypeStruct((D, D), jnp.float32),
    )(btT, itT, utT, w, bias2d)


def _make_sc_lookup(B, T):
    tb = 10
    n_chunk = T // tb
    tt_n = T // 8
    bt_n = B // 128
    mesh = plsc.VectorSubcoreMesh(
        core_axis_name="c", subcore_axis_name="s", num_cores=NC, num_subcores=NS
    )

    @functools.partial(
        pl.kernel,
        mesh=mesh,
        compiler_params=pltpu.CompilerParams(use_tc_tiling_on_sc=False),
        out_type=jax.ShapeDtypeStruct((T, D // 8, bt_n, 8, 128), jnp.float32),
        scratch_types=[
            pltpu.VMEM((tt_n, 1, 8, 128), jnp.int32),
            pltpu.VMEM((tt_n, 1, 8, 128), jnp.int32),
            pltpu.VMEM((tt_n, 1, 8, 128), jnp.int32),
            pltpu.VMEM((2, tb, D // 8, 1, 8, 128), jnp.float32),
            pltpu.VMEM((D, D), jnp.float32),
            pltpu.SemaphoreType.DMA,
            pltpu.SemaphoreType.DMA,
            pltpu.SemaphoreType.DMA,
        ],
    )
    def sc_lookup(f1_hbm, f2_hbm, f3_hbm, ct_hbm, out_hbm,
                  f1_v, f2_v, f3_v, rows_v, ct_v,
                  sem_in, sem_out0, sem_out1):
        wid = lax.axis_index("s") * NC + lax.axis_index("c")
        pltpu.sync_copy(ct_hbm, ct_v)
        cks = [ct_v[k, :] for k in range(D)]
        sems_out = (sem_out0, sem_out1)

        flag_cps = [
            pltpu.async_copy(f_hbm.at[:, pl.ds(wid, 1), :, :], f_v, sem_in)
            for f_hbm, f_v in ((f1_hbm, f1_v), (f2_hbm, f2_v), (f3_hbm, f3_v))
        ]
        for cp in flag_cps:
            cp.wait()

        out_flight = {}
        for ci in range(n_chunk):
            p = ci % 2
            if ci >= 2:
                out_flight.pop(ci - 2).wait()

            def t_body(lt, carry2, ci=ci, p=p):
                t = ci * tb + lt
                tt = t // 8
                tr = t - tt * 8
                for g in range(128 // LANES):
                    s = g * LANES
                    a = f1_v[tt, 0, tr, pl.ds(s, LANES)]
                    bb = f2_v[tt, 0, tr, pl.ds(s, LANES)]
                    cc = f3_v[tt, 0, tr, pl.ds(s, LANES)]
                    code16 = a + bb * 2 + cc * 4
                    for k in range(D):
                        outv = lax.gather(
                            cks[k], code16[:, None], _GATHER_DNUMS, (1,),
                            mode=lax.GatherScatterMode.PROMISE_IN_BOUNDS,
                        )
                        rows_v[p, lt, k // 8, 0, k % 8, pl.ds(s, LANES)] = outv
                return carry2

            lax.fori_loop(0, tb, t_body, 0)
            out_flight[ci] = pltpu.async_copy(
                rows_v.at[p],
                out_hbm.at[pl.ds(ci * tb, tb), :, pl.ds(wid, 1), :, :],
                sems_out[p],
            )
        for cp in out_flight.values():
            cp.wait()

    return sc_lookup


def kernel(bold_flags, italic_flags, underline_flags,
           bold_table, italic_table, underline_table, W, b):
    B, T = bold_flags.shape
    ct = _combo_table_t(
        bold_table.T, italic_table.T, underline_table.T,
        W, b.reshape(D, 1),
    )
    def native_view(f):
        return (f.astype(jnp.int32)
                .reshape(B // 128, 128, T // 8, 8).transpose(2, 0, 3, 1))

    f1 = native_view(bold_flags)
    f2 = native_view(italic_flags)
    f3 = native_view(underline_flags)
    out5 = _make_sc_lookup(B, T)(f1, f2, f3, ct)
    return out5.transpose((2, 4, 0, 1, 3)).reshape(B, T, D)


# speedup vs baseline: 11.2009x; 1.0561x over previous
import functools

import jax
import jax.numpy as jnp
from jax import lax
from jax.experimental import pallas as pl
from jax.experimental.pallas import tpu as pltpu
from jax.experimental.pallas import tpu_sc as plsc

D = 16
NC, NS, LANES = 2, 16, 16
NW = NC * NS
TB = 25

_GATHER_DNUMS = lax.GatherDimensionNumbers(
    offset_dims=(), collapsed_slice_dims=(0,), start_index_map=(0,)
)


def _combo_body(btT, itT, utT, w, bias, c_out):
    code = lax.broadcasted_iota(jnp.int32, (1, 8), 1)
    f1 = (code & 1).astype(jnp.float32)
    f2 = ((code >> 1) & 1).astype(jnp.float32)
    f3 = ((code >> 2) & 1).astype(jnp.float32)
    pb = btT[:, 0:1] + f1 * (btT[:, 1:2] - btT[:, 0:1])
    pi = itT[:, 0:1] + f2 * (itT[:, 1:2] - itT[:, 0:1])
    pu = utT[:, 0:1] + f3 * (utT[:, 1:2] - utT[:, 0:1])
    combT = jnp.concatenate([pb, pi, pu], axis=0)
    ct = jnp.dot(w[...], combT, preferred_element_type=jnp.float32) + bias[...]
    c_out[...] = jnp.concatenate([ct, jnp.zeros((D, 8), jnp.float32)], axis=1)


def _combo_table_t(btT, itT, utT, w, bias2d):
    return pl.pallas_call(
        _combo_body,
        out_shape=jax.ShapeDtypeStruct((D, D), jnp.float32),
    )(btT, itT, utT, w, bias2d)


def _make_sc_lookup(B, T):
    tb = 10
    n_chunk = T // tb
    tt_n = T // 8
    bt_n = B // 128
    mesh = plsc.VectorSubcoreMesh(
        core_axis_name="c", subcore_axis_name="s", num_cores=NC, num_subcores=NS
    )

    ftt = 5
    n_fchunk = tt_n // ftt
    sub_per_f = (ftt * 8) // tb

    @functools.partial(
        pl.kernel,
        mesh=mesh,
        compiler_params=pltpu.CompilerParams(use_tc_tiling_on_sc=False),
        out_type=jax.ShapeDtypeStruct((T, D // 8, bt_n, 8, 128), jnp.float32),
        scratch_types=[
            pltpu.VMEM((2, ftt, 1, 8, 128), jnp.int32),
            pltpu.VMEM((2, ftt, 1, 8, 128), jnp.int32),
            pltpu.VMEM((2, ftt, 1, 8, 128), jnp.int32),
            pltpu.VMEM((2, tb, D // 8, 1, 8, 128), jnp.float32),
            pltpu.VMEM((D, D), jnp.float32),
            pltpu.SemaphoreType.DMA,
            pltpu.SemaphoreType.DMA,
            pltpu.SemaphoreType.DMA,
            pltpu.SemaphoreType.DMA,
        ],
    )
    def sc_lookup(f1_hbm, f2_hbm, f3_hbm, ct_hbm, out_hbm,
                  f1_v, f2_v, f3_v, rows_v, ct_v,
                  sem_in0, sem_in1, sem_out0, sem_out1):
        wid = lax.axis_index("s") * NC + lax.axis_index("c")
        sems_in = (sem_in0, sem_in1)
        sems_out = (sem_out0, sem_out1)
        ct_cp = pltpu.async_copy(ct_hbm, ct_v, sem_in1)

        def start_flags(fi):
            fp = fi % 2
            return [
                pltpu.async_copy(
                    f_hbm.at[pl.ds(fi * ftt, ftt), pl.ds(wid, 1), :, :],
                    f_v.at[fp], sems_in[fp])
                for f_hbm, f_v in ((f1_hbm, f1_v), (f2_hbm, f2_v),
                                   (f3_hbm, f3_v))
            ]

        in_flight = {0: start_flags(0)}
        ct_cp.wait()
        cks = [ct_v[k, :] for k in range(D)]

        out_flight = {}
        for fi in range(n_fchunk):
            fp = fi % 2
            if fi + 1 < n_fchunk:
                in_flight[fi + 1] = start_flags(fi + 1)
            for cp in in_flight.pop(fi):
                cp.wait()
            for sub in range(sub_per_f):
                ci = fi * sub_per_f + sub
                p = ci % 2
                if ci >= 2:
                    out_flight.pop(ci - 2).wait()

                def t_body(lt, carry2, sub=sub, fp=fp, p=p):
                    t = sub * tb + lt
                    tt = t // 8
                    tr = t - tt * 8
                    for g in range(128 // LANES):
                        s = g * LANES
                        a = f1_v[fp, tt, 0, tr, pl.ds(s, LANES)]
                        bb = f2_v[fp, tt, 0, tr, pl.ds(s, LANES)]
                        cc = f3_v[fp, tt, 0, tr, pl.ds(s, LANES)]
                        code16 = a + bb * 2 + cc * 4
                        for k in range(D):
                            outv = lax.gather(
                                cks[k], code16[:, None], _GATHER_DNUMS, (1,),
                                mode=lax.GatherScatterMode.PROMISE_IN_BOUNDS,
                            )
                            rows_v[p, lt, k // 8, 0, k % 8,
                                   pl.ds(s, LANES)] = outv
                    return carry2

                lax.fori_loop(0, tb, t_body, 0)
                out_flight[ci] = pltpu.async_copy(
                    rows_v.at[p],
                    out_hbm.at[pl.ds(ci * tb, tb), :, pl.ds(wid, 1), :, :],
                    sems_out[p],
                )
        for cp in out_flight.values():
            cp.wait()

    return sc_lookup


def kernel(bold_flags, italic_flags, underline_flags,
           bold_table, italic_table, underline_table, W, b):
    B, T = bold_flags.shape
    ct = _combo_table_t(
        bold_table.T, italic_table.T, underline_table.T,
        W, b.reshape(D, 1),
    )
    def native_view(f):
        return (f.astype(jnp.int32)
                .reshape(B // 128, 128, T // 8, 8).transpose(2, 0, 3, 1))

    f1 = native_view(bold_flags)
    f2 = native_view(italic_flags)
    f3 = native_view(underline_flags)
    out5 = _make_sc_lookup(B, T)(f1, f2, f3, ct)
    return out5.transpose((2, 4, 0, 1, 3)).reshape(B, T, D)


# speedup vs baseline: 11.5819x; 1.0340x over previous
import functools

import jax
import jax.numpy as jnp
from jax import lax
from jax.experimental import pallas as pl
from jax.experimental.pallas import tpu as pltpu
from jax.experimental.pallas import tpu_sc as plsc

D = 16
NC, NS, LANES = 2, 16, 16
NW = NC * NS
TB = 25

_GATHER_DNUMS = lax.GatherDimensionNumbers(
    offset_dims=(), collapsed_slice_dims=(0,), start_index_map=(0,)
)


def _combo_body(btT, itT, utT, w, bias, c_out):
    code = lax.broadcasted_iota(jnp.int32, (1, 8), 1)
    f1 = (code & 1).astype(jnp.float32)
    f2 = ((code >> 1) & 1).astype(jnp.float32)
    f3 = ((code >> 2) & 1).astype(jnp.float32)
    pb = btT[:, 0:1] + f1 * (btT[:, 1:2] - btT[:, 0:1])
    pi = itT[:, 0:1] + f2 * (itT[:, 1:2] - itT[:, 0:1])
    pu = utT[:, 0:1] + f3 * (utT[:, 1:2] - utT[:, 0:1])
    combT = jnp.concatenate([pb, pi, pu], axis=0)
    ct = jnp.dot(w[...], combT, preferred_element_type=jnp.float32) + bias[...]
    c_out[...] = jnp.concatenate([ct, jnp.zeros((D, 8), jnp.float32)], axis=1)


def _combo_table_t(btT, itT, utT, w, bias2d):
    return pl.pallas_call(
        _combo_body,
        out_shape=jax.ShapeDtypeStruct((D, D), jnp.float32),
    )(btT, itT, utT, w, bias2d)


def _make_sc_lookup(B, T):
    tb = 20
    n_chunk = T // tb
    tt_n = T // 8
    bt_n = B // 128
    mesh = plsc.VectorSubcoreMesh(
        core_axis_name="c", subcore_axis_name="s", num_cores=NC, num_subcores=NS
    )

    ftt = 5
    n_fchunk = tt_n // ftt
    sub_per_f = (ftt * 8) // tb

    @functools.partial(
        pl.kernel,
        mesh=mesh,
        compiler_params=pltpu.CompilerParams(use_tc_tiling_on_sc=False),
        out_type=jax.ShapeDtypeStruct((T, D // 8, bt_n, 8, 128), jnp.float32),
        scratch_types=[
            pltpu.VMEM((2, ftt, 1, 8, 128), jnp.int32),
            pltpu.VMEM((2, ftt, 1, 8, 128), jnp.int32),
            pltpu.VMEM((2, ftt, 1, 8, 128), jnp.int32),
            pltpu.VMEM((2, tb, D // 8, 1, 8, 128), jnp.float32),
            pltpu.VMEM((D, D), jnp.float32),
            pltpu.SemaphoreType.DMA,
            pltpu.SemaphoreType.DMA,
            pltpu.SemaphoreType.DMA,
            pltpu.SemaphoreType.DMA,
        ],
    )
    def sc_lookup(f1_hbm, f2_hbm, f3_hbm, ct_hbm, out_hbm,
                  f1_v, f2_v, f3_v, rows_v, ct_v,
                  sem_in0, sem_in1, sem_out0, sem_out1):
        wid = lax.axis_index("s") * NC + lax.axis_index("c")
        sems_in = (sem_in0, sem_in1)
        sems_out = (sem_out0, sem_out1)
        ct_cp = pltpu.async_copy(ct_hbm, ct_v, sem_in1)

        def start_flags(fi):
            fp = fi % 2
            return [
                pltpu.async_copy(
                    f_hbm.at[pl.ds(fi * ftt, ftt), pl.ds(wid, 1), :, :],
                    f_v.at[fp], sems_in[fp])
                for f_hbm, f_v in ((f1_hbm, f1_v), (f2_hbm, f2_v),
                                   (f3_hbm, f3_v))
            ]

        in_flight = {0: start_flags(0)}
        ct_cp.wait()
        cks = [ct_v[k, :] for k in range(D)]

        out_flight = {}
        for fi in range(n_fchunk):
            fp = fi % 2
            if fi + 1 < n_fchunk:
                in_flight[fi + 1] = start_flags(fi + 1)
            for cp in in_flight.pop(fi):
                cp.wait()
            for sub in range(sub_per_f):
                ci = fi * sub_per_f + sub
                p = ci % 2
                if ci >= 2:
                    out_flight.pop(ci - 2).wait()

                def t_body(lt, carry2, sub=sub, fp=fp, p=p):
                    t = sub * tb + lt
                    tt = t // 8
                    tr = t - tt * 8
                    for g in range(128 // LANES):
                        s = g * LANES
                        a = f1_v[fp, tt, 0, tr, pl.ds(s, LANES)]
                        bb = f2_v[fp, tt, 0, tr, pl.ds(s, LANES)]
                        cc = f3_v[fp, tt, 0, tr, pl.ds(s, LANES)]
                        code16 = a + bb * 2 + cc * 4
                        for k in range(D):
                            outv = lax.gather(
                                cks[k], code16[:, None], _GATHER_DNUMS, (1,),
                                mode=lax.GatherScatterMode.PROMISE_IN_BOUNDS,
                            )
                            rows_v[p, lt, k // 8, 0, k % 8,
                                   pl.ds(s, LANES)] = outv
                    return carry2

                lax.fori_loop(0, tb, t_body, 0)
                out_flight[ci] = pltpu.async_copy(
                    rows_v.at[p],
                    out_hbm.at[pl.ds(ci * tb, tb), :, pl.ds(wid, 1), :, :],
                    sems_out[p],
                )
        for cp in out_flight.values():
            cp.wait()

    return sc_lookup


def kernel(bold_flags, italic_flags, underline_flags,
           bold_table, italic_table, underline_table, W, b):
    B, T = bold_flags.shape
    ct = _combo_table_t(
        bold_table.T, italic_table.T, underline_table.T,
        W, b.reshape(D, 1),
    )
    def native_view(f):
        return (f.astype(jnp.int32)
                .reshape(B // 128, 128, T // 8, 8).transpose(2, 0, 3, 1))

    f1 = native_view(bold_flags)
    f2 = native_view(italic_flags)
    f3 = native_view(underline_flags)
    out5 = _make_sc_lookup(B, T)(f1, f2, f3, ct)
    return out5.transpose((2, 4, 0, 1, 3)).reshape(B, T, D)


# speedup vs baseline: 12.5139x; 1.0805x over previous
import functools

import jax
import jax.numpy as jnp
from jax import lax
from jax.experimental import pallas as pl
from jax.experimental.pallas import tpu as pltpu
from jax.experimental.pallas import tpu_sc as plsc

D = 16
NC, NS, LANES = 2, 16, 16
NW = NC * NS
TB = 25

_GATHER_DNUMS = lax.GatherDimensionNumbers(
    offset_dims=(), collapsed_slice_dims=(0,), start_index_map=(0,)
)


def _combo_body(btT, itT, utT, w, bias, c_out):
    code = lax.broadcasted_iota(jnp.int32, (1, 8), 1)
    f1 = (code & 1).astype(jnp.float32)
    f2 = ((code >> 1) & 1).astype(jnp.float32)
    f3 = ((code >> 2) & 1).astype(jnp.float32)
    pb = btT[:, 0:1] + f1 * (btT[:, 1:2] - btT[:, 0:1])
    pi = itT[:, 0:1] + f2 * (itT[:, 1:2] - itT[:, 0:1])
    pu = utT[:, 0:1] + f3 * (utT[:, 1:2] - utT[:, 0:1])
    combT = jnp.concatenate([pb, pi, pu], axis=0)
    ct = jnp.dot(w[...], combT, preferred_element_type=jnp.float32) + bias[...]
    c_out[...] = jnp.concatenate([ct, jnp.zeros((D, 8), jnp.float32)], axis=1)


def _combo_table_t(btT, itT, utT, w, bias2d):
    return pl.pallas_call(
        _combo_body,
        out_shape=jax.ShapeDtypeStruct((D, D), jnp.float32),
    )(btT, itT, utT, w, bias2d)


def _make_sc_lookup(B, T):
    tb = 20
    n_chunk = T // tb
    tt_n = T // 8
    bt_n = B // 128
    mesh = plsc.VectorSubcoreMesh(
        core_axis_name="c", subcore_axis_name="s", num_cores=NC, num_subcores=NS
    )

    ftt = 5
    n_fchunk = tt_n // ftt
    sub_per_f = (ftt * 8) // tb

    @functools.partial(
        pl.kernel,
        mesh=mesh,
        compiler_params=pltpu.CompilerParams(use_tc_tiling_on_sc=False),
        out_type=jax.ShapeDtypeStruct((T, D // 8, bt_n, 8, 128), jnp.float32),
        scratch_types=[
            pltpu.VMEM((2, ftt, 1, 8, 128), jnp.int32),
            pltpu.VMEM((2, ftt, 1, 8, 128), jnp.int32),
            pltpu.VMEM((2, ftt, 1, 8, 128), jnp.int32),
            pltpu.VMEM((2, tb, D // 8, 1, 8, 128), jnp.float32),
            pltpu.VMEM((D, D), jnp.float32),
            pltpu.SemaphoreType.DMA,
            pltpu.SemaphoreType.DMA,
            pltpu.SemaphoreType.DMA,
            pltpu.SemaphoreType.DMA,
        ],
    )
    def sc_lookup(f1_hbm, f2_hbm, f3_hbm, ct_hbm, out_hbm,
                  f1_v, f2_v, f3_v, rows_v, ct_v,
                  sem_in0, sem_in1, sem_out0, sem_out1):
        wid = lax.axis_index("s") * NC + lax.axis_index("c")
        sems_in = (sem_in0, sem_in1)
        sems_out = (sem_out0, sem_out1)
        ct_cp = pltpu.async_copy(ct_hbm, ct_v, sem_in1)

        def start_flags(fi):
            fp = fi % 2
            return [
                pltpu.async_copy(
                    f_hbm.at[pl.ds(fi * ftt, ftt), pl.ds(wid, 1), :, :],
                    f_v.at[fp], sems_in[fp])
                for f_hbm, f_v in ((f1_hbm, f1_v), (f2_hbm, f2_v),
                                   (f3_hbm, f3_v))
            ]

        in_flight = {0: start_flags(0)}
        ct_cp.wait()
        cks = [ct_v[k, :] for k in range(D)]

        out_flight = {}
        for fi in range(n_fchunk):
            fp = fi % 2
            if fi + 1 < n_fchunk:
                in_flight[fi + 1] = start_flags(fi + 1)
            for cp in in_flight.pop(fi):
                cp.wait()
            for sub in range(sub_per_f):
                ci = fi * sub_per_f + sub
                p = ci % 2
                if ci >= 2:
                    out_flight.pop(ci - 2).wait()

                @plsc.parallel_loop(0, tb, 1, unroll=2)
                def t_body(lt, sub=sub, fp=fp, p=p):
                    t = sub * tb + lt
                    tt = t // 8
                    tr = t - tt * 8
                    for g in range(128 // LANES):
                        s = g * LANES
                        a = f1_v[fp, tt, 0, tr, pl.ds(s, LANES)]
                        bb = f2_v[fp, tt, 0, tr, pl.ds(s, LANES)]
                        cc = f3_v[fp, tt, 0, tr, pl.ds(s, LANES)]
                        code16 = a + bb * 2 + cc * 4
                        for k in range(D):
                            outv = lax.gather(
                                cks[k], code16[:, None], _GATHER_DNUMS, (1,),
                                mode=lax.GatherScatterMode.PROMISE_IN_BOUNDS,
                            )
                            rows_v[p, lt, k // 8, 0, k % 8,
                                   pl.ds(s, LANES)] = outv
                out_flight[ci] = pltpu.async_copy(
                    rows_v.at[p],
                    out_hbm.at[pl.ds(ci * tb, tb), :, pl.ds(wid, 1), :, :],
                    sems_out[p],
                )
        for cp in out_flight.values():
            cp.wait()

    return sc_lookup


def kernel(bold_flags, italic_flags, underline_flags,
           bold_table, italic_table, underline_table, W, b):
    B, T = bold_flags.shape
    ct = _combo_table_t(
        bold_table.T, italic_table.T, underline_table.T,
        W, b.reshape(D, 1),
    )
    def native_view(f):
        return (f.astype(jnp.int32)
                .reshape(B // 128, 128, T // 8, 8).transpose(2, 0, 3, 1))

    f1 = native_view(bold_flags)
    f2 = native_view(italic_flags)
    f3 = native_view(underline_flags)
    out5 = _make_sc_lookup(B, T)(f1, f2, f3, ct)
    return out5.transpose((2, 4, 0, 1, 3)).reshape(B, T, D)
